# baseline (speedup 1.0000x reference)
import numpy as np
import jax
import jax.numpy as jnp
from jax import lax
from jax.experimental import pallas as pl
from jax.experimental.pallas import tpu as pltpu

B, S, DIM = 20, 500, 300
HID = 600
QK = 128
ROT = 32
NB = 32


G = 2


NSTEP = B // G


def _gau_kernel(x_hbm, emb_ref, bucket_ref, mask2_ref, Wh_ref, bh_ref,
                Wqk_ref, bqk0_ref, gamma_ref, beta_ref,
                lng_ref, lnb_ref,
                cs_ref, sn_ref, R_ref,
                A4_ref, B4_ref, A4T_ref, B4T_ref,
                Wout0_ref, bout_ref, selm_ref, selc_ref, o_hbm,
                xbuf, obuf, sem_in, sem_out,
                bias_s, whv_s, whg_s, wqk_s, wout_s, m2_s):
    f32 = jnp.float32
    bf16 = jnp.bfloat16
    rng = range(G)

    emb = jnp.broadcast_to(emb_ref[...], (S, NB))
    bias_s[...] = jnp.take_along_axis(emb, bucket_ref[...], axis=1)
    whv_s[...] = Wh_ref[:, :HID].astype(bf16)
    whg_s[...] = Wh_ref[:, HID:].astype(bf16)
    wqk_s[...] = Wqk_ref[...].astype(bf16)
    wout_s[...] = Wout0_ref[...].astype(bf16)
    m2_s[...] = (mask2_ref[...] * np.float32(1.0 / (S * S))).astype(bf16)

    lng_b0 = lng_ref[...].astype(bf16)
    lnb_b0 = lnb_ref[...].astype(bf16)
    bhv_r = bh_ref[0:1, :HID].astype(bf16)
    bhg_r = bh_ref[0:1, HID:].astype(bf16)
    bqk_r = bqk0_ref[...].astype(bf16)
    gamma_b = gamma_ref[...].astype(bf16)
    beta_b = beta_ref[...].astype(bf16)

    def in_copy(step, slot):
        return pltpu.make_async_copy(
            x_hbm.at[pl.ds(step * G, G)], xbuf.at[slot], sem_in.at[slot])

    def out_copy(step, slot):
        return pltpu.make_async_copy(
            obuf.at[slot], o_hbm.at[pl.ds(step * G, G)], sem_out.at[slot])

    in_copy(0, 0).start()

    def _step(i, carry):
        slot = jax.lax.rem(i, 2)

        @pl.when(i + 1 < NSTEP)
        def _():
            in_copy(i + 1, 1 - slot).start()

        in_copy(i, slot).wait()

        @pl.when(i >= 2)
        def _():
            out_copy(i - 2, slot).wait()

        _gau_body(xbuf.at[slot], m2_s, bias_s, lng_b0, lnb_b0,
                  whv_s, bhv_r, whg_s, bhg_r,
                  wqk_s, bqk_r, gamma_b, beta_b,
                  cs_ref, sn_ref, R_ref,
                  A4_ref, B4_ref, A4T_ref, B4T_ref,
                  wout_s, bout_ref, obuf.at[slot], selm_ref, selc_ref)
        out_copy(i, slot).start()
        return carry

    lax.fori_loop(0, NSTEP, _step, 0)
    out_copy(NSTEP - 2, 0).wait()
    out_copy(NSTEP - 1, 1).wait()


def _gau_body(x_ref, mask2_ref, bias_ref, lng_b, lnb_b,
              Whv_ref, bhv, Whg_ref, bhg,
              Wqk_ref, bqk2, gamma_b, beta_b,
              cs_ref, sn_ref, R_ref,
              A4_ref, B4_ref, A4T_ref, B4T_ref,
              Wout_ref, bout_ref, o_ref, selm_ref, selc_ref):
    f32 = jnp.float32
    bf16 = jnp.bfloat16
    rng = range(G)
    one_b = jnp.bfloat16(1.0)
    zero_b = jnp.bfloat16(0.0)
    ones_row = jnp.ones((1, S), bf16)

    def _silu(x):
        y = x * 0.5
        return y + y * jnp.tanh(y)

    def _count_ge(absg, thresh_bf):
        geb = jnp.where(absg >= thresh_bf, one_b, zero_b)
        c = jnp.dot(ones_row, geb, preferred_element_type=f32)
        return jnp.sum(c, axis=1, keepdims=True)

    xb = [x_ref[g] for g in rng]
    m = [jnp.mean(xb[g], axis=-1, keepdims=True) for g in rng]
    var = [jnp.mean(xb[g] * xb[g], axis=-1, keepdims=True) - m[g] * m[g]
           for g in rng]
    sc = [lax.rsqrt(var[g] + 1e-5).astype(bf16) for g in rng]
    nxb = [(xb[g].astype(bf16) - m[g].astype(bf16)) * sc[g] * lng_b + lnb_b
           for g in rng]
    nx2b = [jnp.concatenate([jnp.zeros((1, DIM), bf16), nxb[g][:-1]],
                            axis=0) * selm_ref[...] + nxb[g] * selc_ref[...]
            for g in rng]
    v = [jnp.dot(nx2b[g], Whv_ref[...],
                 preferred_element_type=f32).astype(bf16) + bhv
         for g in rng]
    v = [_silu(v[g]) for g in rng]
    gate = [jnp.dot(nx2b[g], Whg_ref[...],
                    preferred_element_type=f32).astype(bf16) + bhg
            for g in rng]
    gate = [_silu(gate[g]) for g in rng]
    absg = [jnp.abs(gate[g]) for g in rng]
    gmax = [jnp.max(jnp.max(gate[g], axis=0, keepdims=True), axis=1,
                    keepdims=True) for g in rng]
    floor_g = [jnp.floor(gmax[g]).astype(f32) for g in rng]

    qk = [jnp.dot(nx2b[g], Wqk_ref[...],
                  preferred_element_type=f32).astype(bf16) + bqk2
          for g in rng]
    qk = [_silu(qk[g]) for g in rng]
    q0 = [qk[g] * gamma_b[0:1, :] + beta_b[0:1, :] for g in rng]
    k0 = [qk[g] * gamma_b[1:2, :] + beta_b[1:2, :] for g in rng]
    cs = cs_ref[...]
    sn = sn_ref[...]
    q = [q0[g] * cs + jnp.dot(q0[g], R_ref[...],
                              preferred_element_type=f32).astype(bf16) * sn
         for g in rng]
    k = [k0[g] * cs + jnp.dot(k0[g], R_ref[...],
                              preferred_element_type=f32).astype(bf16) * sn
         for g in rng]

    cnt1 = [_count_ge(absg[g], one_b) for g in rng]

    def _search_all():
        outs = []
        for g in rng:
            lo = jnp.zeros((1, 1), f32)
            hi = jnp.full((1, 1), 128.0, f32)
            for _ in range(8):
                mid = jnp.floor((lo + hi + 1.0) * 0.5)
                cnt = _count_ge(absg[g], mid.astype(bf16))
                ok = cnt > 90000.0
                lo = jnp.where(ok, mid, lo)
                hi = jnp.where(ok, hi, mid - 1.0)
            outs.append(lo)
        return tuple(outs)

    need = cnt1[0][0, 0]
    for g in range(1, G):
        need = jnp.maximum(need, cnt1[g][0, 0])
    tstar = lax.cond(need > 90000.0, _search_all,
                     lambda: tuple(jnp.zeros((1, 1), f32) for _ in rng))

    trim = [jnp.maximum(jnp.minimum(tstar[g], floor_g[g]), 1.0) for g in rng]
    ind = [jnp.where(absg[g] >= trim[g].astype(bf16), one_b, zero_b)
           for g in rng]
    counts = [jnp.dot(A4_ref[...],
                      jnp.dot(ind[g], B4_ref[...],
                              preferred_element_type=f32).astype(bf16),
                      preferred_element_type=f32) for g in rng]
    cmax = [jnp.max(jnp.max(counts[g], axis=0, keepdims=True), axis=1,
                    keepdims=True) for g in rng]
    t2star = [jnp.zeros((1, 1), f32) for g in rng]
    for t in range(1, 17):
        ct = [jnp.sum(jnp.sum(jnp.where(counts[g] >= float(t), 1.0, 0.0),
                              axis=0, keepdims=True), axis=1, keepdims=True)
              for g in rng]
        t2star = [t2star[g] + jnp.where(ct[g] > 3600.0, 1.0, 0.0) for g in rng]
    t2 = [jnp.where(t2star[g] >= 0.5, t2star[g], cmax[g]) for g in rng]
    bv = [jnp.where(counts[g] >= t2[g], 1.0, 0.25).astype(bf16) for g in rng]
    gm = [jnp.dot(A4T_ref[...],
                  jnp.dot(bv[g], B4T_ref[...],
                          preferred_element_type=f32).astype(bf16),
                  preferred_element_type=f32).astype(bf16) for g in rng]

    sim = [lax.dot_general(q[g], k[g], (((1,), (1,)), ((), ())),
                           preferred_element_type=f32) + bias_ref[...]
           for g in rng]
    a = [jnp.maximum(sim[g].astype(bf16), jnp.bfloat16(0.0)) for g in rng]
    attn = [a[g] * a[g] * mask2_ref[...] for g in rng]
    out = [jnp.dot(attn[g], v[g], preferred_element_type=f32).astype(bf16)
           for g in rng]
    out = [gm[g] * out[g] * gate[g] for g in rng]
    for g in rng:
        o_ref[g] = (jnp.dot(out[g], Wout_ref[...], preferred_element_type=f32)
                    + bout_ref[...] + xb[g])


def kernel(x, my_mask2, ln_g, ln_b, Wh, bh, Wqk, bqk, gamma, beta, rel_emb, Wout, bout):
    f32 = jnp.float32

    inv = 1.0 / (10000.0 ** (jnp.arange(0, ROT, 2, dtype=f32) / ROT))
    fr = jnp.repeat(jnp.arange(S, dtype=f32)[:, None] * inv[None, :], 2, axis=-1)
    cs = jnp.concatenate([jnp.cos(fr), jnp.ones((S, QK - ROT), f32)], axis=1)
    sn = jnp.concatenate([jnp.sin(fr), jnp.zeros((S, QK - ROT), f32)], axis=1)
    Rnp = np.zeros((QK, QK), np.float32)
    for i in range(0, ROT, 2):
        Rnp[i + 1, i] = -1.0
        Rnp[i, i + 1] = 1.0
    R = jnp.asarray(Rnp, jnp.bfloat16)

    A4np = np.zeros((100, S), np.float32)
    for r in range(4):
        A4np[np.arange(100), 5 * np.arange(100) + r] = 1.0
    B4np = np.zeros((HID, 120), np.float32)
    for c in range(4):
        B4np[5 * np.arange(120) + c, np.arange(120)] = 1.0
    A4 = jnp.asarray(A4np, jnp.bfloat16)
    B4 = jnp.asarray(B4np, jnp.bfloat16)
    A4T = jnp.asarray(np.ascontiguousarray(A4np.T), jnp.bfloat16)
    B4T = jnp.asarray(np.ascontiguousarray(B4np.T), jnp.bfloat16)

    nb = NB // 2
    pos = jnp.arange(S)
    n = pos[:, None] - pos[None, :]
    ret = (n < 0).astype(jnp.int32) * nb
    na = jnp.abs(n)
    max_exact = nb // 2
    vil = max_exact + (jnp.log(jnp.maximum(na, 1).astype(f32) / max_exact)
                       / np.float32(np.log(128.0 / max_exact))
                       * (nb - max_exact)).astype(jnp.int32)
    vil = jnp.minimum(vil, nb - 1)
    bucket = (ret + jnp.where(na < max_exact, na, vil)).astype(jnp.int32)

    bf16 = jnp.bfloat16
    emb_row = (rel_emb[:, 0] * np.float32(QK ** 0.5)).reshape(1, NB)
    lng, lnb = ln_g.reshape(1, DIM), ln_b.reshape(1, DIM)
    bout2 = bout.reshape(1, DIM)

    selm_np = np.zeros((1, DIM), np.float32)
    selm_np[0, :DIM // 2] = 1.0
    selm = jnp.asarray(selm_np, bf16)
    selc = jnp.asarray(1.0 - selm_np, bf16)
    consts = (emb_row, bucket, my_mask2, Wh, bh.reshape(1, 2 * HID),
              Wqk, bqk.reshape(1, QK), gamma, beta, lng, lnb,
              cs.astype(bf16), sn.astype(bf16),
              R, A4, B4, A4T, B4T, Wout, bout2, selm, selc)
    in_specs = [pl.BlockSpec(memory_space=pl.ANY)]
    in_specs += [pl.BlockSpec(memory_space=pltpu.VMEM) for _ in consts]

    out = pl.pallas_call(
        _gau_kernel,
        in_specs=in_specs,
        out_specs=pl.BlockSpec(memory_space=pl.ANY),
        out_shape=jax.ShapeDtypeStruct((B, S, DIM), f32),
        scratch_shapes=[
            pltpu.VMEM((2, G, S, DIM), f32),
            pltpu.VMEM((2, G, S, DIM), f32),
            pltpu.SemaphoreType.DMA((2,)),
            pltpu.SemaphoreType.DMA((2,)),
            pltpu.VMEM((S, S), f32),
            pltpu.VMEM((DIM, HID), jnp.bfloat16),
            pltpu.VMEM((DIM, HID), jnp.bfloat16),
            pltpu.VMEM((DIM, QK), jnp.bfloat16),
            pltpu.VMEM((HID, DIM), jnp.bfloat16),
            pltpu.VMEM((S, S), jnp.bfloat16),
        ],
        compiler_params=pltpu.CompilerParams(
            vmem_limit_bytes=48 * 1024 * 1024,
        ),
        name="gau_fused",
    )(x, *consts)
    return out


# speedup vs baseline: 1.0166x; 1.0166x over previous
import numpy as np
import jax
import jax.numpy as jnp
from jax import lax
from jax.experimental import pallas as pl
from jax.experimental.pallas import tpu as pltpu

B, S, DIM = 20, 500, 300
HID = 600
QK = 128
ROT = 32
NB = 32


G = 2


NSTEP = B // G


def _gau_kernel(x_hbm, emb_ref, bucket_ref, mask2_ref, Wh_ref, bh_ref,
                Wqk_ref, bqk0_ref, gamma_ref, beta_ref,
                lng_ref, lnb_ref,
                cs_ref, sn_ref, R_ref,
                A4_ref, B4_ref, A4T_ref, B4T_ref,
                Wout0_ref, bout_ref, o_hbm,
                xbuf, obuf, sem_in, sem_out,
                bias_s, whv_s, whg_s, wqk_s, wout_s, m2_s):
    f32 = jnp.float32
    bf16 = jnp.bfloat16
    rng = range(G)

    emb = jnp.broadcast_to(emb_ref[...], (S, NB))
    bias_s[...] = jnp.take_along_axis(emb, bucket_ref[...], axis=1)
    whv_s[...] = Wh_ref[:, :HID].astype(bf16)
    whg_s[...] = Wh_ref[:, HID:].astype(bf16)
    wqk_s[...] = Wqk_ref[...].astype(bf16)
    wout_s[...] = Wout0_ref[...].astype(bf16)
    m2_s[...] = (mask2_ref[...] * np.float32(1.0 / (S * S))).astype(bf16)

    lng_b0 = lng_ref[...]
    lnb_b0 = lnb_ref[...]
    bhv_r = bh_ref[0:1, :HID].astype(bf16)
    bhg_r = bh_ref[0:1, HID:].astype(bf16)
    bqk_r = bqk0_ref[...].astype(bf16)
    gamma_b = gamma_ref[...].astype(bf16)
    beta_b = beta_ref[...].astype(bf16)

    def in_copy(step, slot):
        return pltpu.make_async_copy(
            x_hbm.at[pl.ds(step * G, G)], xbuf.at[slot], sem_in.at[slot])

    def out_copy(step, slot):
        return pltpu.make_async_copy(
            obuf.at[slot], o_hbm.at[pl.ds(step * G, G)], sem_out.at[slot])

    in_copy(0, 0).start()

    def _step(i, carry):
        slot = jax.lax.rem(i, 2)

        @pl.when(i + 1 < NSTEP)
        def _():
            in_copy(i + 1, 1 - slot).start()

        in_copy(i, slot).wait()

        @pl.when(i >= 2)
        def _():
            out_copy(i - 2, slot).wait()

        _gau_body(xbuf.at[slot], m2_s, bias_s, lng_b0, lnb_b0,
                  whv_s, bhv_r, whg_s, bhg_r,
                  wqk_s, bqk_r, gamma_b, beta_b,
                  cs_ref, sn_ref, R_ref,
                  A4_ref, B4_ref, A4T_ref, B4T_ref,
                  wout_s, bout_ref, obuf.at[slot])
        out_copy(i, slot).start()
        return carry

    lax.fori_loop(0, NSTEP, _step, 0)
    out_copy(NSTEP - 2, 0).wait()
    out_copy(NSTEP - 1, 1).wait()


def _gau_body(x_ref, mask2_ref, bias_ref, lng_b, lnb_b,
              Whv_ref, bhv, Whg_ref, bhg,
              Wqk_ref, bqk2, gamma_b, beta_b,
              cs_ref, sn_ref, R_ref,
              A4_ref, B4_ref, A4T_ref, B4T_ref,
              Wout_ref, bout_ref, o_ref):
    f32 = jnp.float32
    bf16 = jnp.bfloat16
    rng = range(G)
    one_b = jnp.bfloat16(1.0)
    zero_b = jnp.bfloat16(0.0)
    ones_row = jnp.ones((1, S), bf16)

    def _silu(x):
        y = x * 0.5
        return y + y * jnp.tanh(y)

    def _count_ge(absg, thresh_bf):
        geb = jnp.where(absg >= thresh_bf, one_b, zero_b)
        c = jnp.dot(ones_row, geb, preferred_element_type=f32)
        return jnp.sum(c, axis=1, keepdims=True)

    xb = [x_ref[g] for g in rng]
    m = [jnp.mean(xb[g], axis=-1, keepdims=True) for g in rng]
    var = [jnp.mean(xb[g] * xb[g], axis=-1, keepdims=True) - m[g] * m[g]
           for g in rng]
    nx = [(xb[g] - m[g]) * lax.rsqrt(var[g] + 1e-5) * lng_b + lnb_b
          for g in rng]
    lane = lax.broadcasted_iota(jnp.int32, (1, DIM), 1)
    nx2b = [jnp.where(lane < DIM // 2,
                      jnp.concatenate([jnp.zeros((1, DIM), f32), nx[g][:-1]],
                                      axis=0),
                      nx[g]).astype(bf16) for g in rng]
    v = [jnp.dot(nx2b[g], Whv_ref[...],
                 preferred_element_type=f32).astype(bf16) + bhv
         for g in rng]
    v = [_silu(v[g]) for g in rng]
    gate = [jnp.dot(nx2b[g], Whg_ref[...],
                    preferred_element_type=f32).astype(bf16) + bhg
            for g in rng]
    gate = [_silu(gate[g]) for g in rng]
    absg = [jnp.abs(gate[g]) for g in rng]
    gmax = [jnp.max(jnp.max(gate[g], axis=0, keepdims=True), axis=1,
                    keepdims=True) for g in rng]
    floor_g = [jnp.floor(gmax[g]).astype(f32) for g in rng]

    qk = [jnp.dot(nx2b[g], Wqk_ref[...],
                  preferred_element_type=f32).astype(bf16) + bqk2
          for g in rng]
    qk = [_silu(qk[g]) for g in rng]
    q0 = [qk[g] * gamma_b[0:1, :] + beta_b[0:1, :] for g in rng]
    k0 = [qk[g] * gamma_b[1:2, :] + beta_b[1:2, :] for g in rng]
    cs = cs_ref[...]
    sn = sn_ref[...]
    q = [q0[g] * cs + jnp.dot(q0[g], R_ref[...],
                              preferred_element_type=f32).astype(bf16) * sn
         for g in rng]
    k = [k0[g] * cs + jnp.dot(k0[g], R_ref[...],
                              preferred_element_type=f32).astype(bf16) * sn
         for g in rng]

    cnt1 = [_count_ge(absg[g], one_b) for g in rng]

    def _search_all():
        outs = []
        for g in rng:
            lo = jnp.zeros((1, 1), f32)
            hi = jnp.full((1, 1), 128.0, f32)
            for _ in range(8):
                mid = jnp.floor((lo + hi + 1.0) * 0.5)
                cnt = _count_ge(absg[g], mid.astype(bf16))
                ok = cnt > 90000.0
                lo = jnp.where(ok, mid, lo)
                hi = jnp.where(ok, hi, mid - 1.0)
            outs.append(lo)
        return tuple(outs)

    need = cnt1[0][0, 0]
    for g in range(1, G):
        need = jnp.maximum(need, cnt1[g][0, 0])
    tstar = lax.cond(need > 90000.0, _search_all,
                     lambda: tuple(jnp.zeros((1, 1), f32) for _ in rng))

    trim = [jnp.maximum(jnp.minimum(tstar[g], floor_g[g]), 1.0) for g in rng]
    ind = [jnp.where(absg[g] >= trim[g].astype(bf16), one_b, zero_b)
           for g in rng]
    counts = [jnp.dot(A4_ref[...],
                      jnp.dot(ind[g], B4_ref[...],
                              preferred_element_type=f32).astype(bf16),
                      preferred_element_type=f32) for g in rng]
    cmax = [jnp.max(jnp.max(counts[g], axis=0, keepdims=True), axis=1,
                    keepdims=True) for g in rng]
    t2star = [jnp.zeros((1, 1), f32) for g in rng]
    for t in range(1, 17):
        ct = [jnp.sum(jnp.sum(jnp.where(counts[g] >= float(t), 1.0, 0.0),
                              axis=0, keepdims=True), axis=1, keepdims=True)
              for g in rng]
        t2star = [t2star[g] + jnp.where(ct[g] > 3600.0, 1.0, 0.0) for g in rng]
    t2 = [jnp.where(t2star[g] >= 0.5, t2star[g], cmax[g]) for g in rng]
    bv = [jnp.where(counts[g] >= t2[g], 1.0, 0.25).astype(bf16) for g in rng]
    gm = [jnp.dot(A4T_ref[...],
                  jnp.dot(bv[g], B4T_ref[...],
                          preferred_element_type=f32).astype(bf16),
                  preferred_element_type=f32).astype(bf16) for g in rng]

    sim = [lax.dot_general(q[g], k[g], (((1,), (1,)), ((), ())),
                           preferred_element_type=f32) + bias_ref[...]
           for g in rng]
    a = [jnp.maximum(sim[g].astype(bf16), jnp.bfloat16(0.0)) for g in rng]
    attn = [a[g] * a[g] * mask2_ref[...] for g in rng]
    out = [jnp.dot(attn[g], v[g], preferred_element_type=f32).astype(bf16)
           for g in rng]
    out = [gm[g] * out[g] * gate[g] for g in rng]
    for g in rng:
        o_ref[g] = (jnp.dot(out[g], Wout_ref[...], preferred_element_type=f32)
                    + bout_ref[...] + xb[g])


def kernel(x, my_mask2, ln_g, ln_b, Wh, bh, Wqk, bqk, gamma, beta, rel_emb, Wout, bout):
    f32 = jnp.float32

    inv = 1.0 / (10000.0 ** (jnp.arange(0, ROT, 2, dtype=f32) / ROT))
    fr = jnp.repeat(jnp.arange(S, dtype=f32)[:, None] * inv[None, :], 2, axis=-1)
    cs = jnp.concatenate([jnp.cos(fr), jnp.ones((S, QK - ROT), f32)], axis=1)
    sn = jnp.concatenate([jnp.sin(fr), jnp.zeros((S, QK - ROT), f32)], axis=1)
    Rnp = np.zeros((QK, QK), np.float32)
    for i in range(0, ROT, 2):
        Rnp[i + 1, i] = -1.0
        Rnp[i, i + 1] = 1.0
    R = jnp.asarray(Rnp, jnp.bfloat16)

    A4np = np.zeros((100, S), np.float32)
    for r in range(4):
        A4np[np.arange(100), 5 * np.arange(100) + r] = 1.0
    B4np = np.zeros((HID, 120), np.float32)
    for c in range(4):
        B4np[5 * np.arange(120) + c, np.arange(120)] = 1.0
    A4 = jnp.asarray(A4np, jnp.bfloat16)
    B4 = jnp.asarray(B4np, jnp.bfloat16)
    A4T = jnp.asarray(np.ascontiguousarray(A4np.T), jnp.bfloat16)
    B4T = jnp.asarray(np.ascontiguousarray(B4np.T), jnp.bfloat16)

    nb = NB // 2
    pos = jnp.arange(S)
    n = pos[:, None] - pos[None, :]
    ret = (n < 0).astype(jnp.int32) * nb
    na = jnp.abs(n)
    max_exact = nb // 2
    vil = max_exact + (jnp.log(jnp.maximum(na, 1).astype(f32) / max_exact)
                       / np.float32(np.log(128.0 / max_exact))
                       * (nb - max_exact)).astype(jnp.int32)
    vil = jnp.minimum(vil, nb - 1)
    bucket = (ret + jnp.where(na < max_exact, na, vil)).astype(jnp.int32)

    bf16 = jnp.bfloat16
    emb_row = (rel_emb[:, 0] * np.float32(QK ** 0.5)).reshape(1, NB)
    lng, lnb = ln_g.reshape(1, DIM), ln_b.reshape(1, DIM)
    bout2 = bout.reshape(1, DIM)

    consts = (emb_row, bucket, my_mask2, Wh, bh.reshape(1, 2 * HID),
              Wqk, bqk.reshape(1, QK), gamma, beta, lng, lnb,
              cs.astype(bf16), sn.astype(bf16),
              R, A4, B4, A4T, B4T, Wout, bout2)
    in_specs = [pl.BlockSpec(memory_space=pl.ANY)]
    in_specs += [pl.BlockSpec(memory_space=pltpu.VMEM) for _ in consts]

    out = pl.pallas_call(
        _gau_kernel,
        in_specs=in_specs,
        out_specs=pl.BlockSpec(memory_space=pl.ANY),
        out_shape=jax.ShapeDtypeStruct((B, S, DIM), f32),
        scratch_shapes=[
            pltpu.VMEM((2, G, S, DIM), f32),
            pltpu.VMEM((2, G, S, DIM), f32),
            pltpu.SemaphoreType.DMA((2,)),
            pltpu.SemaphoreType.DMA((2,)),
            pltpu.VMEM((S, S), f32),
            pltpu.VMEM((DIM, HID), jnp.bfloat16),
            pltpu.VMEM((DIM, HID), jnp.bfloat16),
            pltpu.VMEM((DIM, QK), jnp.bfloat16),
            pltpu.VMEM((HID, DIM), jnp.bfloat16),
            pltpu.VMEM((S, S), jnp.bfloat16),
        ],
        compiler_params=pltpu.CompilerParams(
            vmem_limit_bytes=48 * 1024 * 1024,
        ),
        name="gau_fused",
    )(x, *consts)
    return out
